# Initial kernel scaffold; baseline (speedup 1.0000x reference)
import functools

import jax
import jax.numpy as jnp
from jax import lax
from jax.experimental import pallas as pl
from jax.experimental.pallas import tpu as pltpu
from jax.experimental.pallas import tpu_sc as plsc

NCORES = 2
NSUB = 16
NW = NCORES * NSUB
CH = 128
LN_EPS = 1e-5


def _make_sc_agg(n, d, chunks, n_pad, compute_deg):
    rows_per_tile = n_pad // NSUB
    out_rows = n // NSUB
    ob_ch = 125
    assert rows_per_tile % CH == 0 and out_rows == 5 * ob_ch

    out_type = [jax.ShapeDtypeStruct((NCORES, n, d), jnp.float32)]
    scratch = [
        pltpu.VMEM_SHARED((n_pad, d), jnp.float32),
        pltpu.VMEM((chunks, CH), jnp.int32),
        pltpu.VMEM((chunks, CH), jnp.int32),
        pltpu.VMEM((CH, d), jnp.float32),
    ]
    if compute_deg:
        out_type.append(jax.ShapeDtypeStruct((NCORES, n, 16), jnp.float32))
        scratch += [
            pltpu.VMEM_SHARED((n_pad, 16), jnp.float32),
            pltpu.VMEM((CH, 16), jnp.float32),
        ]

    mesh = plsc.VectorSubcoreMesh(core_axis_name="core", subcore_axis_name="subcore")

    def body(h_hbm, src_hbm, dst_hbm, *refs):
        if compute_deg:
            agg_out, deg_out, agg_sp, src_v, dst_v, rows_v, deg_sp, ones_v = refs
        else:
            agg_out, agg_sp, src_v, dst_v, rows_v = refs
        c = lax.axis_index("core")
        s = lax.axis_index("subcore")
        w = c * NSUB + s

        pltpu.sync_copy(src_hbm.at[w], src_v)
        pltpu.sync_copy(dst_hbm.at[w], dst_v)

        @pl.loop(0, CH)
        def _(i):
            @pl.loop(0, d // 16)
            def _(k):
                rows_v[i, pl.ds(k * 16, 16)] = jnp.zeros((16,), jnp.float32)

        zbase = s * rows_per_tile

        @pl.loop(0, rows_per_tile // CH)
        def _(k):
            pltpu.sync_copy(rows_v, agg_sp.at[pl.ds(zbase + k * CH, CH)])

        if compute_deg:
            @pl.loop(0, CH)
            def _(i):
                ones_v[i, pl.ds(0, 16)] = jnp.zeros((16,), jnp.float32)

            @pl.loop(0, rows_per_tile // CH)
            def _(k):
                pltpu.sync_copy(ones_v, deg_sp.at[pl.ds(zbase + k * CH, CH)])

            @pl.loop(0, CH)
            def _(i):
                ones_v[i, pl.ds(0, 16)] = jnp.ones((16,), jnp.float32)

        plsc.subcore_barrier()

        @pl.loop(0, chunks)
        def _(j):
            pltpu.sync_copy(h_hbm.at[src_v.at[j]], rows_v)
            pltpu.sync_copy(rows_v, agg_sp.at[dst_v.at[j]], add=True)
            if compute_deg:
                pltpu.sync_copy(ones_v, deg_sp.at[dst_v.at[j]], add=True)

        plsc.subcore_barrier()

        wbase = s * out_rows

        @pl.loop(0, out_rows // ob_ch)
        def _(k):
            pltpu.sync_copy(agg_sp.at[pl.ds(wbase + k * ob_ch, ob_ch)],
                            rows_v.at[pl.ds(0, ob_ch)])
            pltpu.sync_copy(rows_v.at[pl.ds(0, ob_ch)],
                            agg_out.at[c].at[pl.ds(wbase + k * ob_ch, ob_ch)])

        if compute_deg:
            @pl.loop(0, out_rows // ob_ch)
            def _(k):
                pltpu.sync_copy(deg_sp.at[pl.ds(wbase + k * ob_ch, ob_ch)],
                                ones_v.at[pl.ds(0, ob_ch)])
                pltpu.sync_copy(ones_v.at[pl.ds(0, ob_ch)],
                                deg_out.at[c].at[pl.ds(wbase + k * ob_ch, ob_ch)])

    return pl.kernel(body, out_type=out_type, mesh=mesh, scratch_types=scratch)


def _tc_update(h, p, degp, wl, wr, bb, gg, be):
    n, d = h.shape
    r = 1000
    assert n % r == 0

    def body(p_ref, dg_ref, h_ref, wl_ref, wr_ref, b_ref, g_ref, be_ref, o_ref):
        deg = dg_ref[0, :, 0:1] + dg_ref[1, :, 0:1]
        deg = jnp.maximum(deg, 1.0)
        agg = (p_ref[0] + p_ref[1]) / deg
        hh = h_ref[...]
        t = (jnp.dot(agg, wl_ref[...], preferred_element_type=jnp.float32,
                     precision=lax.Precision.HIGHEST)
             + jnp.dot(hh, wr_ref[...], preferred_element_type=jnp.float32,
                       precision=lax.Precision.HIGHEST)
             + b_ref[...])
        mu = jnp.mean(t, axis=-1, keepdims=True)
        var = jnp.mean((t - mu) ** 2, axis=-1, keepdims=True)
        t = (t - mu) * lax.rsqrt(var + LN_EPS) * g_ref[...] + be_ref[...]
        o_ref[...] = hh + jnp.maximum(t, 0.0)

    return pl.pallas_call(
        body,
        grid=(n // r,),
        in_specs=[
            pl.BlockSpec((2, r, d), lambda i: (0, i, 0)),
            pl.BlockSpec((2, r, 16), lambda i: (0, i, 0)),
            pl.BlockSpec((r, d), lambda i: (i, 0)),
            pl.BlockSpec((d, d), lambda i: (0, 0)),
            pl.BlockSpec((d, d), lambda i: (0, 0)),
            pl.BlockSpec((1, d), lambda i: (0, 0)),
            pl.BlockSpec((1, d), lambda i: (0, 0)),
            pl.BlockSpec((1, d), lambda i: (0, 0)),
        ],
        out_specs=pl.BlockSpec((r, d), lambda i: (i, 0)),
        out_shape=jax.ShapeDtypeStruct((n, d), jnp.float32),
    )(p, degp, h, wl, wr, bb, gg, be)


def kernel(node_features, edge_index, Wl, Wr, b, gamma, beta):
    n, d = node_features.shape
    e = edge_index.shape[1]
    l = Wl.shape[0]

    chunks = -(-e // (NW * CH))
    e_pad = NW * chunks * CH
    n_pad = -(-(n + 1) // (NSUB * CH)) * (NSUB * CH)

    src = edge_index[0].astype(jnp.int32)
    dst = edge_index[1].astype(jnp.int32)
    pad = e_pad - e
    src_r = jnp.concatenate([src, jnp.zeros((pad,), jnp.int32)]).reshape(NW, chunks, CH)
    dst_r = jnp.concatenate([dst, jnp.full((pad,), n, jnp.int32)]).reshape(NW, chunks, CH)

    sc_first = _make_sc_agg(n, d, chunks, n_pad, compute_deg=True)
    sc_rest = _make_sc_agg(n, d, chunks, n_pad, compute_deg=False)

    h = node_features
    degp = None
    for i in range(l):
        if i == 0:
            p, degp = sc_first(h, src_r, dst_r)
        else:
            (p,) = sc_rest(h, src_r, dst_r)
        h = _tc_update(h, p, degp, Wl[i], Wr[i],
                       b[i].reshape(1, d), gamma[i].reshape(1, d),
                       beta[i].reshape(1, d))
    return h


# speedup vs baseline: 2.9027x; 2.9027x over previous
import functools

import jax
import jax.numpy as jnp
from jax import lax
from jax.experimental import pallas as pl
from jax.experimental.pallas import tpu as pltpu
from jax.experimental.pallas import tpu_sc as plsc

NCORES = 2
NSUB = 16
NW = NCORES * NSUB
CH = 128
LN_EPS = 1e-5


def _zero_vmem(ref, rows, d):
    @pl.loop(0, rows)
    def _(i):
        @pl.loop(0, d // 16)
        def _(k):
            ref[i, pl.ds(k * 16, 16)] = jnp.zeros((16,), jnp.float32)


def _make_sc_agg(n, d, chunks, n_pad):
    rows_per_tile = n_pad // NSUB
    assert rows_per_tile % CH == 0

    out_type = jax.ShapeDtypeStruct((NCORES, n_pad, d), jnp.float32)
    scratch = [
        pltpu.VMEM_SHARED((n_pad, d), jnp.float32),
        pltpu.VMEM((8, CH), jnp.int32),
        pltpu.VMEM((8, CH), jnp.int32),
        pltpu.VMEM((CH, d), jnp.float32),
    ]
    mesh = plsc.VectorSubcoreMesh(core_axis_name="core", subcore_axis_name="subcore")

    def body(h_hbm, src_hbm, dst_hbm, agg_out, agg_sp, src_v, dst_v, rows_v):
        c = lax.axis_index("core")
        s = lax.axis_index("subcore")
        w = c * NSUB + s

        _zero_vmem(rows_v, CH, d)
        zbase = s * rows_per_tile

        @pl.loop(0, rows_per_tile // CH)
        def _(k):
            pltpu.sync_copy(rows_v, agg_sp.at[pl.ds(zbase + k * CH, CH)])

        plsc.subcore_barrier()

        @pl.loop(0, chunks // 8)
        def _(jb):
            pltpu.sync_copy(src_hbm.at[w].at[pl.ds(jb * 8, 8)], src_v)
            pltpu.sync_copy(dst_hbm.at[w].at[pl.ds(jb * 8, 8)], dst_v)

            @pl.loop(0, 8)
            def _(j):
                pltpu.sync_copy(h_hbm.at[src_v.at[j]], rows_v)
                pltpu.sync_copy(rows_v, agg_sp.at[dst_v.at[j]], add=True)

        plsc.subcore_barrier()

        @pl.loop(0, rows_per_tile // CH)
        def _(k):
            pltpu.sync_copy(agg_sp.at[pl.ds(zbase + k * CH, CH)], rows_v)
            pltpu.sync_copy(rows_v, agg_out.at[c].at[pl.ds(zbase + k * CH, CH)])

    return pl.kernel(body, out_type=out_type, mesh=mesh, scratch_types=scratch)


def _make_sc_deg(n, d, chunks, n_pad):
    rows_per_tile = n_pad // NSUB
    assert rows_per_tile % CH == 0

    out_type = jax.ShapeDtypeStruct((NCORES, n_pad, d), jnp.float32)
    scratch = [
        pltpu.VMEM_SHARED((n_pad, d), jnp.float32),
        pltpu.VMEM((8, CH), jnp.int32),
        pltpu.VMEM((CH, d), jnp.float32),
    ]
    mesh = plsc.VectorSubcoreMesh(core_axis_name="core", subcore_axis_name="subcore")

    def body(dst_hbm, deg_out, deg_sp, dst_v, rows_v):
        c = lax.axis_index("core")
        s = lax.axis_index("subcore")
        w = c * NSUB + s

        _zero_vmem(rows_v, CH, d)
        zbase = s * rows_per_tile

        @pl.loop(0, rows_per_tile // CH)
        def _(k):
            pltpu.sync_copy(rows_v, deg_sp.at[pl.ds(zbase + k * CH, CH)])

        @pl.loop(0, CH)
        def _(i):
            @pl.loop(0, d // 16)
            def _(k):
                rows_v[i, pl.ds(k * 16, 16)] = jnp.ones((16,), jnp.float32)

        plsc.subcore_barrier()

        @pl.loop(0, chunks // 8)
        def _(jb):
            pltpu.sync_copy(dst_hbm.at[w].at[pl.ds(jb * 8, 8)], dst_v)

            @pl.loop(0, 8)
            def _(j):
                pltpu.sync_copy(rows_v, deg_sp.at[dst_v.at[j]], add=True)

        plsc.subcore_barrier()

        @pl.loop(0, rows_per_tile // CH)
        def _(k):
            pltpu.sync_copy(deg_sp.at[pl.ds(zbase + k * CH, CH)], rows_v)
            pltpu.sync_copy(rows_v, deg_out.at[c].at[pl.ds(zbase + k * CH, CH)])

    return pl.kernel(body, out_type=out_type, mesh=mesh, scratch_types=scratch)


def _tc_update(h, p, degp, wl, wr, bb, gg, be):
    n, d = h.shape
    r = 1000
    assert n % r == 0

    def body(p_ref, dg_ref, h_ref, wl_ref, wr_ref, b_ref, g_ref, be_ref, o_ref):
        deg = dg_ref[0, :, 0:1] + dg_ref[1, :, 0:1]
        deg = jnp.maximum(deg, 1.0)
        agg = (p_ref[0] + p_ref[1]) / deg
        hh = h_ref[...]
        t = (jnp.dot(agg, wl_ref[...], preferred_element_type=jnp.float32,
                     precision=lax.Precision.HIGHEST)
             + jnp.dot(hh, wr_ref[...], preferred_element_type=jnp.float32,
                       precision=lax.Precision.HIGHEST)
             + b_ref[...])
        mu = jnp.mean(t, axis=-1, keepdims=True)
        var = jnp.mean((t - mu) ** 2, axis=-1, keepdims=True)
        t = (t - mu) * lax.rsqrt(var + LN_EPS) * g_ref[...] + be_ref[...]
        o_ref[...] = hh + jnp.maximum(t, 0.0)

    return pl.pallas_call(
        body,
        grid=(n // r,),
        in_specs=[
            pl.BlockSpec((2, r, d), lambda i: (0, i, 0)),
            pl.BlockSpec((2, r, 16), lambda i: (0, i, 0)),
            pl.BlockSpec((r, d), lambda i: (i, 0)),
            pl.BlockSpec((d, d), lambda i: (0, 0)),
            pl.BlockSpec((d, d), lambda i: (0, 0)),
            pl.BlockSpec((1, d), lambda i: (0, 0)),
            pl.BlockSpec((1, d), lambda i: (0, 0)),
            pl.BlockSpec((1, d), lambda i: (0, 0)),
        ],
        out_specs=pl.BlockSpec((r, d), lambda i: (i, 0)),
        out_shape=jax.ShapeDtypeStruct((n, d), jnp.float32),
    )(p, degp, h, wl, wr, bb, gg, be)


def kernel(node_features, edge_index, Wl, Wr, b, gamma, beta):
    n, d = node_features.shape
    e = edge_index.shape[1]
    l = Wl.shape[0]

    chunks = -(-(-(-e // (NW * CH))) // 8) * 8
    e_pad = NW * chunks * CH
    n_pad = -(-(n + 1) // (NSUB * CH)) * (NSUB * CH)

    src = edge_index[0].astype(jnp.int32)
    dst = edge_index[1].astype(jnp.int32)
    pad = e_pad - e
    src_r = jnp.concatenate([src, jnp.zeros((pad,), jnp.int32)]).reshape(NW, chunks, CH)
    dst_r = jnp.concatenate([dst, jnp.full((pad,), n, jnp.int32)]).reshape(NW, chunks, CH)

    sc_agg = _make_sc_agg(n, d, chunks, n_pad)
    sc_deg = _make_sc_deg(n, d, chunks, n_pad)

    degp = sc_deg(dst_r)[:, :, :16]

    h = node_features
    for i in range(l):
        p = sc_agg(h, src_r, dst_r)
        h = _tc_update(h, p, degp, Wl[i], Wr[i],
                       b[i].reshape(1, d), gamma[i].reshape(1, d),
                       beta[i].reshape(1, d))
    return h


# speedup vs baseline: 3.4002x; 1.1714x over previous
import functools

import jax
import jax.numpy as jnp
from jax import lax
from jax.experimental import pallas as pl
from jax.experimental.pallas import tpu as pltpu
from jax.experimental.pallas import tpu_sc as plsc

NCORES = 2
NSUB = 16
NW = NCORES * NSUB
CH = 128
LN_EPS = 1e-5


def _row_chunks(total):
    out, off = [], 0
    while off < total:
        sz = min(CH, total - off)
        out.append((off, sz))
        off += sz
    return out


def _zero_vmem(ref, rows, d, value=0.0):
    @pl.loop(0, rows)
    def _(i):
        @pl.loop(0, d // 16)
        def _(k):
            ref[i, pl.ds(k * 16, 16)] = jnp.full((16,), value, jnp.float32)


def _make_sc_agg(n, d, chunks, n_pad):
    rows_per_tile = n_pad // NSUB
    assert rows_per_tile % 8 == 0 and chunks % 8 == 0

    out_type = jax.ShapeDtypeStruct((NCORES, n_pad, d), jnp.float32)
    scratch = [
        pltpu.VMEM_SHARED((n_pad, d), jnp.float32),
        pltpu.VMEM((8, CH), jnp.int32),
        pltpu.VMEM((8, CH), jnp.int32),
        pltpu.VMEM((CH, d), jnp.float32),
        pltpu.VMEM((CH, d), jnp.float32),
        pltpu.SemaphoreType.DMA,
        pltpu.SemaphoreType.DMA,
    ]
    mesh = plsc.VectorSubcoreMesh(core_axis_name="core", subcore_axis_name="subcore")

    def body(h_hbm, src_hbm, dst_hbm, agg_out,
             agg_sp, src_v, dst_v, rows0, rows1, sem0, sem1):
        c = lax.axis_index("core")
        s = lax.axis_index("subcore")
        w = c * NSUB + s
        rows = (rows0, rows1)
        sems = (sem0, sem1)
        zbase = s * rows_per_tile

        _zero_vmem(rows0, CH, d)
        for off, sz in _row_chunks(rows_per_tile):
            pltpu.sync_copy(rows0.at[pl.ds(0, sz)],
                            agg_sp.at[pl.ds(zbase + off, sz)])

        plsc.subcore_barrier()

        def wait(b):
            pltpu.make_async_copy(h_hbm.at[pl.ds(0, CH)], rows[b], sems[b]).wait()

        @pl.loop(0, chunks // 8)
        def _(jb):
            pltpu.sync_copy(src_hbm.at[w].at[pl.ds(jb * 8, 8)], src_v)
            pltpu.sync_copy(dst_hbm.at[w].at[pl.ds(jb * 8, 8)], dst_v)
            pltpu.async_copy(h_hbm.at[src_v.at[0]], rows0, sem0)
            for t in range(4):
                c0, c1 = 2 * t, 2 * t + 1
                pltpu.async_copy(h_hbm.at[src_v.at[c1]], rows1, sem1)
                wait(0)
                pltpu.sync_copy(rows0, agg_sp.at[dst_v.at[c0]], add=True)
                if c1 + 1 < 8:
                    pltpu.async_copy(h_hbm.at[src_v.at[c1 + 1]], rows0, sem0)
                wait(1)
                pltpu.sync_copy(rows1, agg_sp.at[dst_v.at[c1]], add=True)

        plsc.subcore_barrier()

        for off, sz in _row_chunks(rows_per_tile):
            pltpu.sync_copy(agg_sp.at[pl.ds(zbase + off, sz)],
                            rows0.at[pl.ds(0, sz)])
            pltpu.sync_copy(rows0.at[pl.ds(0, sz)],
                            agg_out.at[c].at[pl.ds(zbase + off, sz)])

    return pl.kernel(body, out_type=out_type, mesh=mesh, scratch_types=scratch)


def _make_sc_deg(n, d, chunks, n_pad):
    rows_per_tile = n_pad // NSUB
    assert rows_per_tile % 8 == 0 and chunks % 8 == 0

    out_type = jax.ShapeDtypeStruct((NCORES, n_pad, d), jnp.float32)
    scratch = [
        pltpu.VMEM_SHARED((n_pad, d), jnp.float32),
        pltpu.VMEM((8, CH), jnp.int32),
        pltpu.VMEM((CH, d), jnp.float32),
    ]
    mesh = plsc.VectorSubcoreMesh(core_axis_name="core", subcore_axis_name="subcore")

    def body(dst_hbm, deg_out, deg_sp, dst_v, rows_v):
        c = lax.axis_index("core")
        s = lax.axis_index("subcore")
        w = c * NSUB + s
        zbase = s * rows_per_tile

        _zero_vmem(rows_v, CH, d)
        for off, sz in _row_chunks(rows_per_tile):
            pltpu.sync_copy(rows_v.at[pl.ds(0, sz)],
                            deg_sp.at[pl.ds(zbase + off, sz)])

        _zero_vmem(rows_v, CH, d, value=1.0)

        plsc.subcore_barrier()

        @pl.loop(0, chunks // 8)
        def _(jb):
            pltpu.sync_copy(dst_hbm.at[w].at[pl.ds(jb * 8, 8)], dst_v)

            @pl.loop(0, 8)
            def _(j):
                pltpu.sync_copy(rows_v, deg_sp.at[dst_v.at[j]], add=True)

        plsc.subcore_barrier()

        for off, sz in _row_chunks(rows_per_tile):
            pltpu.sync_copy(deg_sp.at[pl.ds(zbase + off, sz)],
                            rows_v.at[pl.ds(0, sz)])
            pltpu.sync_copy(rows_v.at[pl.ds(0, sz)],
                            deg_out.at[c].at[pl.ds(zbase + off, sz)])

    return pl.kernel(body, out_type=out_type, mesh=mesh, scratch_types=scratch)


def _tc_update(h, p, degp, wl, wr, bb, gg, be):
    n, d = h.shape
    r = 1000
    assert n % r == 0

    def body(p_ref, dg_ref, h_ref, wl_ref, wr_ref, b_ref, g_ref, be_ref, o_ref):
        deg = dg_ref[0, :, 0:1] + dg_ref[1, :, 0:1]
        deg = jnp.maximum(deg, 1.0)
        agg = (p_ref[0] + p_ref[1]) / deg
        hh = h_ref[...]
        t = (jnp.dot(agg, wl_ref[...], preferred_element_type=jnp.float32,
                     precision=lax.Precision.HIGHEST)
             + jnp.dot(hh, wr_ref[...], preferred_element_type=jnp.float32,
                       precision=lax.Precision.HIGHEST)
             + b_ref[...])
        mu = jnp.mean(t, axis=-1, keepdims=True)
        var = jnp.mean((t - mu) ** 2, axis=-1, keepdims=True)
        t = (t - mu) * lax.rsqrt(var + LN_EPS) * g_ref[...] + be_ref[...]
        o_ref[...] = hh + jnp.maximum(t, 0.0)

    return pl.pallas_call(
        body,
        grid=(n // r,),
        in_specs=[
            pl.BlockSpec((2, r, d), lambda i: (0, i, 0)),
            pl.BlockSpec((2, r, 16), lambda i: (0, i, 0)),
            pl.BlockSpec((r, d), lambda i: (i, 0)),
            pl.BlockSpec((d, d), lambda i: (0, 0)),
            pl.BlockSpec((d, d), lambda i: (0, 0)),
            pl.BlockSpec((1, d), lambda i: (0, 0)),
            pl.BlockSpec((1, d), lambda i: (0, 0)),
            pl.BlockSpec((1, d), lambda i: (0, 0)),
        ],
        out_specs=pl.BlockSpec((r, d), lambda i: (i, 0)),
        out_shape=jax.ShapeDtypeStruct((n, d), jnp.float32),
    )(p, degp, h, wl, wr, bb, gg, be)


def kernel(node_features, edge_index, Wl, Wr, b, gamma, beta):
    n, d = node_features.shape
    e = edge_index.shape[1]
    l = Wl.shape[0]

    chunks = -(-(-(-e // (NW * CH))) // 8) * 8
    e_pad = NW * chunks * CH
    n_pad = -(-(n + 1) // (NSUB * 8)) * (NSUB * 8)

    src = edge_index[0].astype(jnp.int32)
    dst = edge_index[1].astype(jnp.int32)
    pad = e_pad - e
    src_r = jnp.concatenate([src, jnp.zeros((pad,), jnp.int32)]).reshape(NW, chunks, CH)
    dst_r = jnp.concatenate([dst, jnp.full((pad,), n, jnp.int32)]).reshape(NW, chunks, CH)

    sc_agg = _make_sc_agg(n, d, chunks, n_pad)
    sc_deg = _make_sc_deg(n, d, chunks, n_pad)

    degp = sc_deg(dst_r)[:, :, :16]

    h = node_features
    for i in range(l):
        p = sc_agg(h, src_r, dst_r)
        h = _tc_update(h, p, degp, Wl[i], Wr[i],
                       b[i].reshape(1, d), gamma[i].reshape(1, d),
                       beta[i].reshape(1, d))
    return h
